# Initial kernel scaffold; baseline (speedup 1.0000x reference)
import jax
import jax.numpy as jnp
from jax.experimental import pallas as pl


def kernel(x, edge_index, W, bias, u):
    raise NotImplementedError("write your pallas kernel here")


# speedup vs baseline: 13.8223x; 13.8223x over previous
import functools

import jax
import jax.numpy as jnp
from jax import lax
from jax.experimental import pallas as pl
from jax.experimental.pallas import tpu as pltpu
from jax.experimental.pallas import tpu_sc as plsc

NC = 2
NS = 16
CHUNK = 128


def _sc_mesh():
    return plsc.VectorSubcoreMesh(
        core_axis_name="c", subcore_axis_name="s", num_cores=NC, num_subcores=NS
    )


def _deg_body(pt, acc_rows, n, w, col2d, ones_hbm, zeros_hbm, out_hbm,
              degacc, cidx_v, ones_v):
    c = lax.axis_index("c")
    s = lax.axis_index("s")
    tid = c * NS + s
    zrows = acc_rows // NS
    for k in range(zrows // CHUNK):
        pltpu.sync_copy(zeros_hbm, degacc.at[pl.ds(s * zrows + k * CHUNK, CHUNK)])
    pltpu.sync_copy(ones_hbm, ones_v)
    plsc.subcore_barrier()
    base = tid * pt

    def group(g, carry):
        pltpu.sync_copy(col2d.at[pl.ds(base + g * GRP, GRP)], cidx_v)

        def chunk(i, carry2):
            pltpu.sync_copy(ones_v, degacc.at[cidx_v.at[i]], add=True)
            return carry2

        lax.fori_loop(0, GRP, chunk, 0)
        return carry

    lax.fori_loop(0, pt // GRP, group, 0)
    plsc.subcore_barrier()
    pltpu.sync_copy(degacc.at[pl.ds(s * zrows, zrows)],
                    out_hbm.at[c, pl.ds(s * zrows, zrows)])


GRP = 8


def _acc_body(pt, acc_rows, n, f, row2d, col2d, tmp_hbm, zeros_hbm, out_hbm,
              acc, ridx_v, cidx_v, rows_a, rows_b, sem_a, sem_b):
    c = lax.axis_index("c")
    s = lax.axis_index("s")
    tid = c * NS + s
    zrows = acc_rows // NS
    for k in range(zrows // CHUNK):
        pltpu.sync_copy(zeros_hbm, acc.at[pl.ds(s * zrows + k * CHUNK, CHUNK)])
    plsc.subcore_barrier()
    base = tid * pt

    def group(g, carry):
        off = base + g * GRP
        pltpu.sync_copy(row2d.at[pl.ds(off, GRP)], ridx_v)
        pltpu.sync_copy(col2d.at[pl.ds(off, GRP)], cidx_v)
        pltpu.async_copy(tmp_hbm.at[ridx_v.at[0]], rows_a, sem_a)

        def body(i, carry2):
            ca = 2 * i
            cb = 2 * i + 1
            pltpu.async_copy(tmp_hbm.at[ridx_v.at[cb]], rows_b, sem_b)
            pltpu.make_async_copy(tmp_hbm.at[ridx_v.at[ca]], rows_a, sem_a).wait()
            pltpu.sync_copy(rows_a, acc.at[cidx_v.at[ca]], add=True)

            @pl.when(cb + 1 < GRP)
            def _():
                pltpu.async_copy(tmp_hbm.at[ridx_v.at[cb + 1]], rows_a, sem_a)

            pltpu.make_async_copy(tmp_hbm.at[ridx_v.at[cb]], rows_b, sem_b).wait()
            pltpu.sync_copy(rows_b, acc.at[cidx_v.at[cb]], add=True)
            return carry2

        lax.fori_loop(0, GRP // 2, body, 0)
        return carry

    lax.fori_loop(0, pt // GRP, group, 0)
    plsc.subcore_barrier()
    pltpu.sync_copy(acc.at[pl.ds(s * zrows, zrows)],
                    out_hbm.at[c, pl.ds(s * zrows, zrows)])


def _enc_body(x_ref, w_ref, u_ref, d0_ref, d1_ref, tmp_ref, dis_ref, wsn_ref):
    @pl.when(pl.program_id(0) == 0)
    def _():
        wm = w_ref[...]
        uc = u_ref[...]
        v = lax.dot_general(wm, uc, (((0,), (0,)), ((), ())),
                            preferred_element_type=jnp.float32)
        v = v / (jnp.sqrt(jnp.sum(v * v)) + 1e-12)
        wv = lax.dot_general(wm, v, (((1,), (0,)), ((), ())),
                             preferred_element_type=jnp.float32)
        nwv = jnp.sqrt(jnp.sum(wv * wv))
        sigma = jnp.sum((wv / (nwv + 1e-12)) * wv)
        wsn_ref[...] = wm / sigma

    h = jnp.dot(x_ref[...], wsn_ref[...], preferred_element_type=jnp.float32)
    deg = d0_ref[...] + d1_ref[...] + 1.0
    dis = lax.rsqrt(deg)
    dis_ref[...] = dis
    tmp_ref[...] = h * dis


def _out_body(tmp_ref, a0_ref, a1_ref, dis_ref, b_ref, o_ref):
    o_ref[...] = ((tmp_ref[...] + a0_ref[...] + a1_ref[...]) * dis_ref[...]
                  + b_ref[...])


def kernel(x, edge_index, W, bias, u):
    n, f_in = x.shape
    f_out = W.shape[1]
    e = edge_index.shape[1]
    assert n % NS == 0

    ntiles = NC * NS
    e_pad = -(-e // (ntiles * CHUNK * 8)) * (ntiles * CHUNK * 8)
    pt = e_pad // (ntiles * CHUNK)
    acc_rows = (n // (NS * CHUNK) + 1) * NS * CHUNK

    row = edge_index[0]
    col = edge_index[1]
    pad = e_pad - e
    rowp = jnp.concatenate([row, jnp.zeros((pad,), row.dtype)])
    colp = jnp.concatenate([col, jnp.full((pad,), n, col.dtype)])
    row2d = rowp.reshape(e_pad // CHUNK, CHUNK)
    col2d = colp.reshape(e_pad // CHUNK, CHUNK)

    ones128 = jnp.ones((CHUNK, f_out), jnp.float32)
    zeros128 = jnp.zeros((CHUNK, f_out), jnp.float32)

    degout = pl.kernel(
        functools.partial(_deg_body, pt, acc_rows, n, f_out),
        out_type=jax.ShapeDtypeStruct((NC, acc_rows, f_out), jnp.float32),
        mesh=_sc_mesh(),
        scratch_types=[
            pltpu.VMEM_SHARED((acc_rows, f_out), jnp.float32),
            pltpu.VMEM((GRP, CHUNK), jnp.int32),
            pltpu.VMEM((CHUNK, f_out), jnp.float32),
        ],
    )(col2d, ones128, zeros128)

    d0 = degout[0, :n, 0:1]
    d1 = degout[1, :n, 0:1]

    br = 2000 if n % 2000 == 0 else n // 8
    tmp, dis = pl.pallas_call(
        _enc_body,
        grid=(n // br,),
        in_specs=[
            pl.BlockSpec((br, f_in), lambda i: (i, 0)),
            pl.BlockSpec((f_in, f_out), lambda i: (0, 0)),
            pl.BlockSpec((f_in, 1), lambda i: (0, 0)),
            pl.BlockSpec((br, 1), lambda i: (i, 0)),
            pl.BlockSpec((br, 1), lambda i: (i, 0)),
        ],
        out_specs=[
            pl.BlockSpec((br, f_out), lambda i: (i, 0)),
            pl.BlockSpec((br, 1), lambda i: (i, 0)),
        ],
        out_shape=[
            jax.ShapeDtypeStruct((n, f_out), jnp.float32),
            jax.ShapeDtypeStruct((n, 1), jnp.float32),
        ],
        scratch_shapes=[pltpu.VMEM((f_in, f_out), jnp.float32)],
    )(x, W, u.reshape(f_in, 1), d0, d1)

    accout = pl.kernel(
        functools.partial(_acc_body, pt, acc_rows, n, f_out),
        out_type=jax.ShapeDtypeStruct((NC, acc_rows, f_out), jnp.float32),
        mesh=_sc_mesh(),
        scratch_types=[
            pltpu.VMEM_SHARED((acc_rows, f_out), jnp.float32),
            pltpu.VMEM((GRP, CHUNK), jnp.int32),
            pltpu.VMEM((GRP, CHUNK), jnp.int32),
            pltpu.VMEM((CHUNK, f_out), jnp.float32),
            pltpu.VMEM((CHUNK, f_out), jnp.float32),
            pltpu.SemaphoreType.DMA,
            pltpu.SemaphoreType.DMA,
        ],
    )(row2d, col2d, tmp, zeros128)

    out = pl.pallas_call(
        _out_body,
        grid=(n // br,),
        in_specs=[
            pl.BlockSpec((br, f_out), lambda i: (i, 0)),
            pl.BlockSpec((br, f_out), lambda i: (i, 0)),
            pl.BlockSpec((br, f_out), lambda i: (i, 0)),
            pl.BlockSpec((br, 1), lambda i: (i, 0)),
            pl.BlockSpec((1, f_out), lambda i: (0, 0)),
        ],
        out_specs=pl.BlockSpec((br, f_out), lambda i: (i, 0)),
        out_shape=jax.ShapeDtypeStruct((n, f_out), jnp.float32),
    )(tmp, accout[0, :n], accout[1, :n], dis, bias.reshape(1, f_out))

    return out


# speedup vs baseline: 16.2811x; 1.1779x over previous
import functools

import jax
import jax.numpy as jnp
from jax import lax
from jax.experimental import pallas as pl
from jax.experimental.pallas import tpu as pltpu
from jax.experimental.pallas import tpu_sc as plsc

NC = 2
NS = 16
CHUNK = 128


def _sc_mesh():
    return plsc.VectorSubcoreMesh(
        core_axis_name="c", subcore_axis_name="s", num_cores=NC, num_subcores=NS
    )


DEG_W = 8


def _deg_body(pt, acc_rows, n, col2d, ones_hbm, zeros_hbm, out_hbm,
              degacc, cidx_v, ones_v, sem):
    c = lax.axis_index("c")
    s = lax.axis_index("s")
    tid = c * NS + s
    zrows = acc_rows // NS
    pltpu.sync_copy(zeros_hbm, degacc.at[pl.ds(s * zrows, zrows)])
    pltpu.sync_copy(ones_hbm, ones_v)
    plsc.subcore_barrier()
    base = tid * pt

    def group(g, carry):
        pltpu.sync_copy(col2d.at[pl.ds(base + g * GRP, GRP)], cidx_v)
        descs = [pltpu.async_copy(ones_v, degacc.at[cidx_v.at[i]], sem, add=True)
                 for i in range(GRP)]
        for d in descs:
            d.wait()
        return carry

    lax.fori_loop(0, pt // GRP, group, 0)
    plsc.subcore_barrier()
    pltpu.sync_copy(degacc.at[pl.ds(s * zrows, zrows)],
                    out_hbm.at[c, pl.ds(s * zrows, zrows)])


GRP = 8


def _acc_body(pt, acc_rows, n, f, row2d, col2d, tmp_hbm, zeros_hbm, out_hbm,
              acc, ridx_v, cidx_v, rows_a, rows_b, sem_g, sem_s):
    c = lax.axis_index("c")
    s = lax.axis_index("s")
    tid = c * NS + s
    zrows = acc_rows // NS
    for k in range(zrows // CHUNK):
        pltpu.sync_copy(zeros_hbm, acc.at[pl.ds(s * zrows + k * CHUNK, CHUNK)])
    plsc.subcore_barrier()
    base = tid * pt
    ngroups = pt // GRP
    bufs = (rows_a, rows_b)

    def scatter_wait():
        pltpu.make_async_copy(zeros_hbm, rows_a, sem_s).wait()

    pltpu.sync_copy(row2d.at[pl.ds(base, GRP)], ridx_v.at[0])
    pltpu.sync_copy(col2d.at[pl.ds(base, GRP)], cidx_v.at[0])
    pltpu.async_copy(tmp_hbm.at[ridx_v.at[0, 0]], rows_a, sem_g)

    def group(g, carry):
        gb = lax.rem(g, 2)
        gb1 = lax.rem(g + 1, 2)

        @pl.when(g + 1 < ngroups)
        def _():
            off = base + (g + 1) * GRP
            pltpu.sync_copy(row2d.at[pl.ds(off, GRP)], ridx_v.at[gb1])
            pltpu.sync_copy(col2d.at[pl.ds(off, GRP)], cidx_v.at[gb1])

        for k in range(GRP):
            if k == 0:
                @pl.when(g > 0)
                def _():
                    scatter_wait()
            else:
                scatter_wait()
            if k < GRP - 1:
                pltpu.async_copy(tmp_hbm.at[ridx_v.at[gb, k + 1]],
                                 bufs[(k + 1) % 2], sem_g)
            else:
                @pl.when(g + 1 < ngroups)
                def _():
                    pltpu.async_copy(tmp_hbm.at[ridx_v.at[gb1, 0]],
                                     bufs[0], sem_g)
            pltpu.make_async_copy(tmp_hbm.at[ridx_v.at[gb, k]],
                                  bufs[k % 2], sem_g).wait()
            pltpu.async_copy(bufs[k % 2], acc.at[cidx_v.at[gb, k]], sem_s,
                             add=True)
        return carry

    lax.fori_loop(0, ngroups, group, 0)
    scatter_wait()
    plsc.subcore_barrier()
    pltpu.sync_copy(acc.at[pl.ds(s * zrows, zrows)],
                    out_hbm.at[c, pl.ds(s * zrows, zrows)])


def _enc_body(x_ref, w_ref, u_ref, d0_ref, d1_ref, tmp_ref, dis_ref, wsn_ref):
    @pl.when(pl.program_id(0) == 0)
    def _():
        wm = w_ref[...]
        uc = u_ref[...]
        v = lax.dot_general(wm, uc, (((0,), (0,)), ((), ())),
                            preferred_element_type=jnp.float32)
        v = v / (jnp.sqrt(jnp.sum(v * v)) + 1e-12)
        wv = lax.dot_general(wm, v, (((1,), (0,)), ((), ())),
                             preferred_element_type=jnp.float32)
        nwv = jnp.sqrt(jnp.sum(wv * wv))
        sigma = jnp.sum((wv / (nwv + 1e-12)) * wv)
        wsn_ref[...] = wm / sigma

    h = jnp.dot(x_ref[...], wsn_ref[...], preferred_element_type=jnp.float32)
    deg = d0_ref[...] + d1_ref[...] + 1.0
    dis = lax.rsqrt(deg)
    dis_ref[...] = dis
    tmp_ref[...] = h * dis


def _out_body(tmp_ref, a0_ref, a1_ref, dis_ref, b_ref, o_ref):
    o_ref[...] = ((tmp_ref[...] + a0_ref[0] + a1_ref[0]) * dis_ref[...]
                  + b_ref[...])


def kernel(x, edge_index, W, bias, u):
    n, f_in = x.shape
    f_out = W.shape[1]
    e = edge_index.shape[1]
    assert n % NS == 0

    ntiles = NC * NS
    e_pad = -(-e // (ntiles * CHUNK * 8)) * (ntiles * CHUNK * 8)
    pt = e_pad // (ntiles * CHUNK)
    acc_rows = (n // (NS * CHUNK) + 1) * NS * CHUNK

    row = edge_index[0]
    col = edge_index[1]
    pad = e_pad - e
    rowp = jnp.concatenate([row, jnp.zeros((pad,), row.dtype)])
    colp = jnp.concatenate([col, jnp.full((pad,), n, col.dtype)])
    row2d = rowp.reshape(e_pad // CHUNK, CHUNK)
    col2d = colp.reshape(e_pad // CHUNK, CHUNK)

    ones8 = jnp.ones((CHUNK, DEG_W), jnp.float32)
    zeros8 = jnp.zeros((acc_rows // NS, DEG_W), jnp.float32)
    zeros128 = jnp.zeros((CHUNK, f_out), jnp.float32)

    degout = pl.kernel(
        functools.partial(_deg_body, pt, acc_rows, n),
        out_type=jax.ShapeDtypeStruct((NC, acc_rows, DEG_W), jnp.float32),
        mesh=_sc_mesh(),
        scratch_types=[
            pltpu.VMEM_SHARED((acc_rows, DEG_W), jnp.float32),
            pltpu.VMEM((GRP, CHUNK), jnp.int32),
            pltpu.VMEM((CHUNK, DEG_W), jnp.float32),
            pltpu.SemaphoreType.DMA,
        ],
        compiler_params=pltpu.CompilerParams(use_tc_tiling_on_sc=False),
    )(col2d, ones8, zeros8)

    d0 = degout[0, :n, 0:1]
    d1 = degout[1, :n, 0:1]

    br = 2000 if n % 2000 == 0 else n // 8
    tmp, dis = pl.pallas_call(
        _enc_body,
        grid=(n // br,),
        in_specs=[
            pl.BlockSpec((br, f_in), lambda i: (i, 0)),
            pl.BlockSpec((f_in, f_out), lambda i: (0, 0)),
            pl.BlockSpec((f_in, 1), lambda i: (0, 0)),
            pl.BlockSpec((br, 1), lambda i: (i, 0)),
            pl.BlockSpec((br, 1), lambda i: (i, 0)),
        ],
        out_specs=[
            pl.BlockSpec((br, f_out), lambda i: (i, 0)),
            pl.BlockSpec((br, 1), lambda i: (i, 0)),
        ],
        out_shape=[
            jax.ShapeDtypeStruct((n, f_out), jnp.float32),
            jax.ShapeDtypeStruct((n, 1), jnp.float32),
        ],
        scratch_shapes=[pltpu.VMEM((f_in, f_out), jnp.float32)],
    )(x, W, u.reshape(f_in, 1), d0, d1)

    accout = pl.kernel(
        functools.partial(_acc_body, pt, acc_rows, n, f_out),
        out_type=jax.ShapeDtypeStruct((NC, acc_rows, f_out), jnp.float32),
        mesh=_sc_mesh(),
        scratch_types=[
            pltpu.VMEM_SHARED((acc_rows, f_out), jnp.float32),
            pltpu.VMEM((2, GRP, CHUNK), jnp.int32),
            pltpu.VMEM((2, GRP, CHUNK), jnp.int32),
            pltpu.VMEM((CHUNK, f_out), jnp.float32),
            pltpu.VMEM((CHUNK, f_out), jnp.float32),
            pltpu.SemaphoreType.DMA,
            pltpu.SemaphoreType.DMA,
        ],
    )(row2d, col2d, tmp, zeros128)

    out = pl.pallas_call(
        _out_body,
        grid=(n // br,),
        in_specs=[
            pl.BlockSpec((br, f_out), lambda i: (i, 0)),
            pl.BlockSpec((1, br, f_out), lambda i: (0, i, 0)),
            pl.BlockSpec((1, br, f_out), lambda i: (1, i, 0)),
            pl.BlockSpec((br, 1), lambda i: (i, 0)),
            pl.BlockSpec((1, f_out), lambda i: (0, 0)),
        ],
        out_specs=pl.BlockSpec((br, f_out), lambda i: (i, 0)),
        out_shape=jax.ShapeDtypeStruct((n, f_out), jnp.float32),
    )(tmp, accout, accout, dis, bias.reshape(1, f_out))

    return out


# speedup vs baseline: 16.5609x; 1.0172x over previous
import functools

import jax
import jax.numpy as jnp
from jax import lax
from jax.experimental import pallas as pl
from jax.experimental.pallas import tpu as pltpu
from jax.experimental.pallas import tpu_sc as plsc

NC = 2
NS = 16
CHUNK = 128


def _sc_mesh():
    return plsc.VectorSubcoreMesh(
        core_axis_name="c", subcore_axis_name="s", num_cores=NC, num_subcores=NS
    )


DEG_W = 8


def _deg_body(pt, acc_rows, n, col2d, ones_hbm, zeros_hbm, out_hbm,
              degacc, cidx_v, ones_v, sem):
    c = lax.axis_index("c")
    s = lax.axis_index("s")
    tid = c * NS + s
    zrows = acc_rows // NS
    pltpu.sync_copy(zeros_hbm, degacc.at[pl.ds(s * zrows, zrows)])
    pltpu.sync_copy(ones_hbm, ones_v)
    plsc.subcore_barrier()
    base = tid * pt

    def group(g, carry):
        pltpu.sync_copy(col2d.at[pl.ds(base + g * GRP, GRP)], cidx_v)
        descs = [pltpu.async_copy(ones_v, degacc.at[cidx_v.at[i]], sem, add=True)
                 for i in range(GRP)]
        for d in descs:
            d.wait()
        return carry

    lax.fori_loop(0, pt // GRP, group, 0)
    plsc.subcore_barrier()
    pltpu.sync_copy(degacc.at[pl.ds(s * zrows, zrows)],
                    out_hbm.at[c, pl.ds(s * zrows, zrows)])


GRP = 8


def _acc_body(p0, p1, acc_rows, n, f, row2d, col2d, tmp_hbm, zeros_hbm, out_hbm,
              acc, ridx_v, cidx_v, rows_a, rows_b, sem_g, sem_s):
    c = lax.axis_index("c")
    s = lax.axis_index("s")
    zrows = acc_rows // NS
    for k in range(zrows // CHUNK):
        pltpu.sync_copy(zeros_hbm, acc.at[pl.ds(s * zrows + k * CHUNK, CHUNK)])
    plsc.subcore_barrier()
    base = jnp.where(c == 0, s * p0, NS * p0 + s * p1)
    ngroups = jnp.where(c == 0, p0 // GRP, p1 // GRP)
    bufs = (rows_a, rows_b)

    def scatter_wait():
        pltpu.make_async_copy(zeros_hbm, rows_a, sem_s).wait()

    pltpu.sync_copy(row2d.at[pl.ds(base, GRP)], ridx_v.at[0])
    pltpu.sync_copy(col2d.at[pl.ds(base, GRP)], cidx_v.at[0])
    pltpu.async_copy(tmp_hbm.at[ridx_v.at[0, 0]], rows_a, sem_g)

    def group(g, carry):
        gb = lax.rem(g, 2)
        gb1 = lax.rem(g + 1, 2)

        @pl.when(g + 1 < ngroups)
        def _():
            off = base + (g + 1) * GRP
            pltpu.sync_copy(row2d.at[pl.ds(off, GRP)], ridx_v.at[gb1])
            pltpu.sync_copy(col2d.at[pl.ds(off, GRP)], cidx_v.at[gb1])

        for k in range(GRP):
            if k == 0:
                @pl.when(g > 0)
                def _():
                    scatter_wait()
            else:
                scatter_wait()
            if k < GRP - 1:
                pltpu.async_copy(tmp_hbm.at[ridx_v.at[gb, k + 1]],
                                 bufs[(k + 1) % 2], sem_g)
            else:
                @pl.when(g + 1 < ngroups)
                def _():
                    pltpu.async_copy(tmp_hbm.at[ridx_v.at[gb1, 0]],
                                     bufs[0], sem_g)
            pltpu.make_async_copy(tmp_hbm.at[ridx_v.at[gb, k]],
                                  bufs[k % 2], sem_g).wait()
            pltpu.async_copy(bufs[k % 2], acc.at[cidx_v.at[gb, k]], sem_s,
                             add=True)
        return carry

    lax.fori_loop(0, ngroups, group, 0)
    scatter_wait()
    plsc.subcore_barrier()
    pltpu.sync_copy(acc.at[pl.ds(s * zrows, zrows)],
                    out_hbm.at[c, pl.ds(s * zrows, zrows)])


def _enc_body(x_ref, w_ref, u_ref, d0_ref, d1_ref, tmp_ref, dis_ref, wsn_ref):
    @pl.when(pl.program_id(0) == 0)
    def _():
        wm = w_ref[...]
        uc = u_ref[...]
        v = lax.dot_general(wm, uc, (((0,), (0,)), ((), ())),
                            preferred_element_type=jnp.float32)
        v = v / (jnp.sqrt(jnp.sum(v * v)) + 1e-12)
        wv = lax.dot_general(wm, v, (((1,), (0,)), ((), ())),
                             preferred_element_type=jnp.float32)
        nwv = jnp.sqrt(jnp.sum(wv * wv))
        sigma = jnp.sum((wv / (nwv + 1e-12)) * wv)
        wsn_ref[...] = wm / sigma

    h = jnp.dot(x_ref[...], wsn_ref[...], preferred_element_type=jnp.float32)
    deg = d0_ref[...] + d1_ref[...] + 1.0
    dis = lax.rsqrt(deg)
    dis_ref[...] = dis
    tmp_ref[...] = h * dis


def _out_body(tmp_ref, a0_ref, a1_ref, dis_ref, b_ref, o_ref):
    o_ref[...] = ((tmp_ref[...] + a0_ref[0] + a1_ref[0]) * dis_ref[...]
                  + b_ref[...])


def kernel(x, edge_index, W, bias, u):
    n, f_in = x.shape
    f_out = W.shape[1]
    e = edge_index.shape[1]
    assert n % NS == 0

    ntiles = NC * NS
    e_pad = -(-e // (ntiles * CHUNK * 8)) * (ntiles * CHUNK * 8)
    pt = e_pad // (ntiles * CHUNK)
    acc_rows = (n // (NS * CHUNK) + 1) * NS * CHUNK

    row = edge_index[0]
    col = edge_index[1]
    pad = e_pad - e
    rowp = jnp.concatenate([row, jnp.zeros((pad,), row.dtype)])
    colp = jnp.concatenate([col, jnp.full((pad,), n, col.dtype)])
    row2d = rowp.reshape(e_pad // CHUNK, CHUNK)
    col2d = colp.reshape(e_pad // CHUNK, CHUNK)

    ones8 = jnp.ones((CHUNK, DEG_W), jnp.float32)
    zeros8 = jnp.zeros((acc_rows // NS, DEG_W), jnp.float32)
    zeros128 = jnp.zeros((CHUNK, f_out), jnp.float32)

    degout = pl.kernel(
        functools.partial(_deg_body, pt, acc_rows, n),
        out_type=jax.ShapeDtypeStruct((NC, acc_rows, DEG_W), jnp.float32),
        mesh=_sc_mesh(),
        scratch_types=[
            pltpu.VMEM_SHARED((acc_rows, DEG_W), jnp.float32),
            pltpu.VMEM((GRP, CHUNK), jnp.int32),
            pltpu.VMEM((CHUNK, DEG_W), jnp.float32),
            pltpu.SemaphoreType.DMA,
        ],
        compiler_params=pltpu.CompilerParams(use_tc_tiling_on_sc=False),
    )(col2d, ones8, zeros8)

    d0 = degout[0, :n, 0:1]
    d1 = degout[1, :n, 0:1]

    br = 2000 if n % 2000 == 0 else n // 8
    tmp, dis = pl.pallas_call(
        _enc_body,
        grid=(n // br,),
        in_specs=[
            pl.BlockSpec((br, f_in), lambda i: (i, 0)),
            pl.BlockSpec((f_in, f_out), lambda i: (0, 0)),
            pl.BlockSpec((f_in, 1), lambda i: (0, 0)),
            pl.BlockSpec((br, 1), lambda i: (i, 0)),
            pl.BlockSpec((br, 1), lambda i: (i, 0)),
        ],
        out_specs=[
            pl.BlockSpec((br, f_out), lambda i: (i, 0)),
            pl.BlockSpec((br, 1), lambda i: (i, 0)),
        ],
        out_shape=[
            jax.ShapeDtypeStruct((n, f_out), jnp.float32),
            jax.ShapeDtypeStruct((n, 1), jnp.float32),
        ],
        scratch_shapes=[pltpu.VMEM((f_in, f_out), jnp.float32)],
    )(x, W, u.reshape(f_in, 1), d0, d1)

    total_chunks = e_pad // CHUNK
    p0 = (3 * total_chunks // 4) // (NS * GRP) * GRP
    p1 = total_chunks // NS - p0
    accout = pl.kernel(
        functools.partial(_acc_body, p0, p1, acc_rows, n, f_out),
        out_type=jax.ShapeDtypeStruct((NC, acc_rows, f_out), jnp.float32),
        mesh=_sc_mesh(),
        scratch_types=[
            pltpu.VMEM_SHARED((acc_rows, f_out), jnp.float32),
            pltpu.VMEM((2, GRP, CHUNK), jnp.int32),
            pltpu.VMEM((2, GRP, CHUNK), jnp.int32),
            pltpu.VMEM((CHUNK, f_out), jnp.float32),
            pltpu.VMEM((CHUNK, f_out), jnp.float32),
            pltpu.SemaphoreType.DMA,
            pltpu.SemaphoreType.DMA,
        ],
    )(row2d, col2d, tmp, zeros128)

    out = pl.pallas_call(
        _out_body,
        grid=(n // br,),
        in_specs=[
            pl.BlockSpec((br, f_out), lambda i: (i, 0)),
            pl.BlockSpec((1, br, f_out), lambda i: (0, i, 0)),
            pl.BlockSpec((1, br, f_out), lambda i: (1, i, 0)),
            pl.BlockSpec((br, 1), lambda i: (i, 0)),
            pl.BlockSpec((1, f_out), lambda i: (0, 0)),
        ],
        out_specs=pl.BlockSpec((br, f_out), lambda i: (i, 0)),
        out_shape=jax.ShapeDtypeStruct((n, f_out), jnp.float32),
    )(tmp, accout, accout, dis, bias.reshape(1, f_out))

    return out
